# baseline (speedup 1.0000x reference)
import functools

import jax
import jax.numpy as jnp
from jax import lax
from jax.experimental import pallas as pl
from jax.experimental.pallas import tpu as pltpu
from jax.experimental.pallas import tpu_sc as plsc

N = 10000
D = 128
H = 256
HH = H // 2
E = 320000

NC = 2
NS = 16
CH = 88
NCH = 232
BLK = 8
NBLK = NCH // BLK
NBUF = 4
EPT = NCH * CH
EPAD = EPT * NS

NP = N + 8
INITR = 632
OUTR = 632

_f32 = jnp.float32
_i32 = jnp.int32


def _sc_mesh():
    return plsc.VectorSubcoreMesh(core_axis_name="c", subcore_axis_name="s")


def _agg_body(y_hbm, src_hbm, dst_hbm, out_hbm, sidx, didx, rows0, rows1,
              rows2, rows3, acc, sem0, sem1, sem2, sem3):
    c = lax.axis_index("c")
    s = lax.axis_index("s")
    ibase = c * NP + jnp.minimum(s * INITR, NP - INITR)
    iacc = jnp.minimum(s * INITR, NP - INITR)
    pltpu.sync_copy(y_hbm.at[pl.ds(ibase, INITR)], acc.at[pl.ds(iacc, INITR)])
    plsc.subcore_barrier()

    rows = (rows0, rows1, rows2, rows3)
    sems = (sem0, sem1, sem2, sem3)

    def _block(bi, carry):
        pltpu.sync_copy(src_hbm.at[c, s, pl.ds(bi * BLK, BLK)], sidx)
        pltpu.sync_copy(dst_hbm.at[s, pl.ds(bi * BLK, BLK)], didx)
        descs = [None] * BLK
        for j in range(NBUF):
            descs[j] = pltpu.async_copy(y_hbm.at[sidx.at[j]], rows[j % NBUF],
                                        sems[j % NBUF])
        for j in range(BLK):
            descs[j].wait()
            pltpu.sync_copy(rows[j % NBUF], acc.at[didx.at[j]], add=True)
            if j + NBUF < BLK:
                descs[j + NBUF] = pltpu.async_copy(
                    y_hbm.at[sidx.at[j + NBUF]], rows[j % NBUF],
                    sems[j % NBUF])
        return carry

    lax.fori_loop(0, NBLK, _block, 0)
    plsc.subcore_barrier()
    obase = jnp.minimum(s * OUTR, N - OUTR)
    pltpu.sync_copy(acc.at[pl.ds(obase, OUTR)],
                    out_hbm.at[pl.ds(c * N + obase, OUTR)])


_agg_call = functools.partial(
    pl.kernel,
    out_type=jax.ShapeDtypeStruct((NC * N, HH), _f32),
    mesh=_sc_mesh(),
    scratch_types=[
        pltpu.VMEM((BLK, CH), _i32),
        pltpu.VMEM((BLK, CH), _i32),
        pltpu.VMEM((CH, HH), _f32),
        pltpu.VMEM((CH, HH), _f32),
        pltpu.VMEM((CH, HH), _f32),
        pltpu.VMEM((CH, HH), _f32),
        pltpu.VMEM_SHARED((NP, HH), _f32),
        pltpu.SemaphoreType.DMA,
        pltpu.SemaphoreType.DMA,
        pltpu.SemaphoreType.DMA,
        pltpu.SemaphoreType.DMA,
    ],
)(_agg_body)


def _deg_body(y_hbm, dst_hbm, out_hbm, didx, rows, acc):
    c = lax.axis_index("c")
    s = lax.axis_index("s")
    ibase = c * NP + jnp.minimum(s * INITR, NP - INITR)
    iacc = jnp.minimum(s * INITR, NP - INITR)
    pltpu.sync_copy(y_hbm.at[pl.ds(ibase, INITR)], acc.at[pl.ds(iacc, INITR)])
    pltpu.sync_copy(y_hbm.at[pl.ds(0, CH)], rows)
    plsc.subcore_barrier()

    def _block(bi, carry):
        pltpu.sync_copy(dst_hbm.at[s, pl.ds(bi * BLK, BLK)], didx)

        def _chunk(j, carry2):
            pltpu.sync_copy(rows, acc.at[didx.at[j]], add=True)
            return carry2

        lax.fori_loop(0, BLK, _chunk, 0)
        return carry

    lax.fori_loop(0, NBLK, _block, 0)
    plsc.subcore_barrier()
    obase = jnp.minimum(s * OUTR, N - OUTR)
    pltpu.sync_copy(acc.at[pl.ds(obase, OUTR)],
                    out_hbm.at[pl.ds(c * N + obase, OUTR)])


_deg_call = functools.partial(
    pl.kernel,
    out_type=jax.ShapeDtypeStruct((NC * N, HH), _f32),
    mesh=_sc_mesh(),
    scratch_types=[
        pltpu.VMEM((BLK, CH), _i32),
        pltpu.VMEM((CH, HH), _f32),
        pltpu.VMEM_SHARED((NP, HH), _f32),
    ],
)(_deg_body)


def _first_body(x_ref, w_ref, deg_ref, y_ref, dinv_ref):
    dinv = lax.rsqrt(deg_ref[...])
    xw = jnp.dot(x_ref[...], w_ref[...], preferred_element_type=_f32)
    y = xw * dinv
    y_ref[:N, :] = y[:, :HH]
    y_ref[NP:NP + N, :] = y[:, HH:]
    dinv_ref[...] = dinv


_first_call = pl.pallas_call(
    _first_body,
    out_shape=(
        jax.ShapeDtypeStruct((NC * NP, HH), _f32),
        jax.ShapeDtypeStruct((N, 1), _f32),
    ),
)


def _mid_body(a_ref, dinv_ref, b_ref, w_ref, y_ref):
    dinv = dinv_ref[...]
    b = b_ref[...]
    h0 = jnp.maximum(a_ref[:N] * dinv + b[:HH], 0.0)
    h1 = jnp.maximum(a_ref[N:] * dinv + b[HH:], 0.0)
    y = (jnp.dot(h0, w_ref[:HH, :], preferred_element_type=_f32)
         + jnp.dot(h1, w_ref[HH:, :], preferred_element_type=_f32)) * dinv
    y_ref[:N, :] = y[:, :HH]
    y_ref[NP:NP + N, :] = y[:, HH:]


_mid_call = pl.pallas_call(
    _mid_body,
    out_shape=jax.ShapeDtypeStruct((NC * NP, HH), _f32),
)


def _head_body(a_ref, dinv_ref, b_ref, x_ref, fw1_ref, fb1_ref, fw2_ref,
               fb2_ref, iw_ref, ib_ref, mw_ref, mb_ref, is_ref, mc_ref):
    dinv = dinv_ref[...]
    b = b_ref[...]
    h0 = jnp.maximum(a_ref[:N] * dinv + b[:HH], 0.0)
    h1 = jnp.maximum(a_ref[N:] * dinv + b[HH:], 0.0)
    m0 = jnp.mean(h0, axis=0, keepdims=True)
    m1 = jnp.mean(h1, axis=0, keepdims=True)

    xv = x_ref[...]
    col = lambda i: xv[:, i:i + 1]
    ssum = lambda v: jnp.sum(v, axis=0, keepdims=True)
    n_comp = ssum(col(2))
    n_and = ssum(col(3))
    n_or = ssum(col(4))
    comp = col(2) == 1.0
    cnt = ssum(comp.astype(_f32))
    sum_l = ssum(jnp.where(comp, col(0), 0.0))
    sum_m = ssum(jnp.where(comp, col(1), 0.0))
    avg_l = jnp.where(cnt > 0, sum_l / jnp.maximum(cnt, 1.0), 0.0)
    avg_m = jnp.where(cnt > 0, sum_m / jnp.maximum(cnt, 1.0), 0.0)
    t_norm = jnp.full((1, 1), 100.0 / 500.0, _f32)
    t_fac = jnp.full((1, 1), 1.0 + 6.0 * (1.0 - 100.0 / 500.0) ** 1.5, _f32)
    gf = jnp.concatenate(
        [n_comp, n_and, n_or, n_and + n_or, avg_l, avg_m, t_norm, t_fac],
        axis=1)

    emb = jnp.concatenate([m0, m1, gf], axis=1)
    f = jnp.maximum(
        jnp.dot(emb, fw1_ref[...], preferred_element_type=_f32) + fb1_ref[...],
        0.0)
    f = jnp.maximum(
        jnp.dot(f, fw2_ref[...], preferred_element_type=_f32) + fb2_ref[...],
        0.0)
    is_ref[...] = jnp.dot(f, iw_ref[...], preferred_element_type=_f32) + ib_ref[...]
    mc_ref[...] = jnp.dot(f, mw_ref[...], preferred_element_type=_f32) + mb_ref[...]


_head_call = pl.pallas_call(
    _head_body,
    out_shape=(
        jax.ShapeDtypeStruct((1, 6), _f32),
        jax.ShapeDtypeStruct((1, 6), _f32),
    ),
)


def kernel(x, edge_index, W1, b1, W2, b2, W3, b3, fcW1, fcb1, fcW2, fcb2,
           isW, isb, mcW, mcb):
    pad = jnp.full((EPAD - E,), N, _i32)
    srcr = jnp.concatenate([edge_index[0], pad]).reshape(NS, NCH, CH)
    dstr = jnp.concatenate([edge_index[1], pad]).reshape(NS, NCH, CH)
    srcc = jnp.stack([srcr, srcr + NP])

    ones_y = jnp.ones((NC * NP, HH), _f32)
    deg_col = _deg_call(ones_y, dstr)[:N, 0:1]

    y, dinv = _first_call(x, W1, deg_col)
    agg = _agg_call(y, srcc, dstr)
    y = _mid_call(agg, dinv, b1, W2)
    agg = _agg_call(y, srcc, dstr)
    y = _mid_call(agg, dinv, b2, W3)
    agg = _agg_call(y, srcc, dstr)
    return _head_call(agg, dinv, b3, x, fcW1, fcb1, fcW2, fcb2,
                      isW, isb, mcW, mcb)


# speedup vs baseline: 1.6306x; 1.6306x over previous
import functools

import jax
import jax.numpy as jnp
from jax import lax
from jax.experimental import pallas as pl
from jax.experimental.pallas import tpu as pltpu
from jax.experimental.pallas import tpu_sc as plsc

N = 10000
D = 128
H = 256
HH = H // 2
E = 320000

NC = 2
NS = 16
CH = 120
NCH = 168
BLK = 8
NBLK = NCH // BLK
NBUF = 3
EPT = NCH * CH
EPAD = EPT * NS

NP = N + 8
INITR = 632
OUTR = 632

_f32 = jnp.float32
_i32 = jnp.int32


def _sc_mesh():
    return plsc.VectorSubcoreMesh(core_axis_name="c", subcore_axis_name="s")


def _agg_body(y_hbm, src_hbm, dst_hbm, out_hbm, sidx, didx, rows0, rows1,
              rows2, acc, sem0, sem1, sem2):
    c = lax.axis_index("c")
    s = lax.axis_index("s")
    ibase = c * NP + jnp.minimum(s * INITR, NP - INITR)
    iacc = jnp.minimum(s * INITR, NP - INITR)
    pltpu.sync_copy(y_hbm.at[pl.ds(ibase, INITR)], acc.at[pl.ds(iacc, INITR)])
    plsc.subcore_barrier()

    rows = (rows0, rows1, rows2)
    sems = (sem0, sem1, sem2)

    def _block(bi, carry):
        pltpu.sync_copy(src_hbm.at[c, s, pl.ds(bi * BLK, BLK)], sidx)
        pltpu.sync_copy(dst_hbm.at[s, pl.ds(bi * BLK, BLK)], didx)
        descs = [None] * BLK
        for j in range(NBUF):
            descs[j] = pltpu.async_copy(y_hbm.at[sidx.at[j]], rows[j % NBUF],
                                        sems[j % NBUF])
        for j in range(BLK):
            descs[j].wait()
            pltpu.sync_copy(rows[j % NBUF], acc.at[didx.at[j]], add=True)
            if j + NBUF < BLK:
                descs[j + NBUF] = pltpu.async_copy(
                    y_hbm.at[sidx.at[j + NBUF]], rows[j % NBUF],
                    sems[j % NBUF])
        return carry

    lax.fori_loop(0, NBLK, _block, 0)
    plsc.subcore_barrier()
    obase = jnp.minimum(s * OUTR, N - OUTR)
    pltpu.sync_copy(acc.at[pl.ds(obase, OUTR)],
                    out_hbm.at[pl.ds(c * N + obase, OUTR)])


_agg_call = functools.partial(
    pl.kernel,
    out_type=jax.ShapeDtypeStruct((NC * N, HH), _f32),
    mesh=_sc_mesh(),
    scratch_types=[
        pltpu.VMEM((BLK, CH), _i32),
        pltpu.VMEM((BLK, CH), _i32),
        pltpu.VMEM((CH, HH), _f32),
        pltpu.VMEM((CH, HH), _f32),
        pltpu.VMEM((CH, HH), _f32),
        pltpu.VMEM_SHARED((NP, HH), _f32),
        pltpu.SemaphoreType.DMA,
        pltpu.SemaphoreType.DMA,
        pltpu.SemaphoreType.DMA,
    ],
)(_agg_body)


def _deg_body(y_hbm, dst_hbm, out_hbm, didx, rows, acc):
    c = lax.axis_index("c")
    s = lax.axis_index("s")
    ibase = c * NP + jnp.minimum(s * INITR, NP - INITR)
    iacc = jnp.minimum(s * INITR, NP - INITR)
    pltpu.sync_copy(y_hbm.at[pl.ds(ibase, INITR)], acc.at[pl.ds(iacc, INITR)])
    pltpu.sync_copy(y_hbm.at[pl.ds(0, CH)], rows)
    plsc.subcore_barrier()

    def _block(bi, carry):
        pltpu.sync_copy(dst_hbm.at[s, pl.ds(bi * BLK, BLK)], didx)

        def _chunk(j, carry2):
            pltpu.sync_copy(rows, acc.at[didx.at[j]], add=True)
            return carry2

        lax.fori_loop(0, BLK, _chunk, 0)
        return carry

    lax.fori_loop(0, NBLK, _block, 0)
    plsc.subcore_barrier()
    obase = jnp.minimum(s * OUTR, N - OUTR)
    pltpu.sync_copy(acc.at[pl.ds(obase, OUTR)],
                    out_hbm.at[pl.ds(c * N + obase, OUTR)])


_deg_call = functools.partial(
    pl.kernel,
    out_type=jax.ShapeDtypeStruct((NC * N, HH), _f32),
    mesh=_sc_mesh(),
    scratch_types=[
        pltpu.VMEM((BLK, CH), _i32),
        pltpu.VMEM((CH, HH), _f32),
        pltpu.VMEM_SHARED((NP, HH), _f32),
    ],
)(_deg_body)


def _first_body(x_ref, w_ref, deg_ref, y_ref, dinv_ref):
    dinv = lax.rsqrt(deg_ref[...])
    xw = jnp.dot(x_ref[...], w_ref[...], preferred_element_type=_f32)
    y = xw * dinv
    y_ref[:N, :] = y[:, :HH]
    y_ref[NP:NP + N, :] = y[:, HH:]
    dinv_ref[...] = dinv


_first_call = pl.pallas_call(
    _first_body,
    out_shape=(
        jax.ShapeDtypeStruct((NC * NP, HH), _f32),
        jax.ShapeDtypeStruct((N, 1), _f32),
    ),
)


def _mid_body(a_ref, dinv_ref, b_ref, w_ref, y_ref):
    dinv = dinv_ref[...]
    b = b_ref[...]
    h0 = jnp.maximum(a_ref[:N] * dinv + b[:HH], 0.0)
    h1 = jnp.maximum(a_ref[N:] * dinv + b[HH:], 0.0)
    y = (jnp.dot(h0, w_ref[:HH, :], preferred_element_type=_f32)
         + jnp.dot(h1, w_ref[HH:, :], preferred_element_type=_f32)) * dinv
    y_ref[:N, :] = y[:, :HH]
    y_ref[NP:NP + N, :] = y[:, HH:]


_mid_call = pl.pallas_call(
    _mid_body,
    out_shape=jax.ShapeDtypeStruct((NC * NP, HH), _f32),
)


def _head_body(a_ref, dinv_ref, b_ref, x_ref, fw1_ref, fb1_ref, fw2_ref,
               fb2_ref, iw_ref, ib_ref, mw_ref, mb_ref, is_ref, mc_ref):
    dinv = dinv_ref[...]
    b = b_ref[...]
    h0 = jnp.maximum(a_ref[:N] * dinv + b[:HH], 0.0)
    h1 = jnp.maximum(a_ref[N:] * dinv + b[HH:], 0.0)
    m0 = jnp.mean(h0, axis=0, keepdims=True)
    m1 = jnp.mean(h1, axis=0, keepdims=True)

    xv = x_ref[...]
    col = lambda i: xv[:, i:i + 1]
    ssum = lambda v: jnp.sum(v, axis=0, keepdims=True)
    n_comp = ssum(col(2))
    n_and = ssum(col(3))
    n_or = ssum(col(4))
    comp = col(2) == 1.0
    cnt = ssum(comp.astype(_f32))
    sum_l = ssum(jnp.where(comp, col(0), 0.0))
    sum_m = ssum(jnp.where(comp, col(1), 0.0))
    avg_l = jnp.where(cnt > 0, sum_l / jnp.maximum(cnt, 1.0), 0.0)
    avg_m = jnp.where(cnt > 0, sum_m / jnp.maximum(cnt, 1.0), 0.0)
    t_norm = jnp.full((1, 1), 100.0 / 500.0, _f32)
    t_fac = jnp.full((1, 1), 1.0 + 6.0 * (1.0 - 100.0 / 500.0) ** 1.5, _f32)
    gf = jnp.concatenate(
        [n_comp, n_and, n_or, n_and + n_or, avg_l, avg_m, t_norm, t_fac],
        axis=1)

    emb = jnp.concatenate([m0, m1, gf], axis=1)
    f = jnp.maximum(
        jnp.dot(emb, fw1_ref[...], preferred_element_type=_f32) + fb1_ref[...],
        0.0)
    f = jnp.maximum(
        jnp.dot(f, fw2_ref[...], preferred_element_type=_f32) + fb2_ref[...],
        0.0)
    is_ref[...] = jnp.dot(f, iw_ref[...], preferred_element_type=_f32) + ib_ref[...]
    mc_ref[...] = jnp.dot(f, mw_ref[...], preferred_element_type=_f32) + mb_ref[...]


_head_call = pl.pallas_call(
    _head_body,
    out_shape=(
        jax.ShapeDtypeStruct((1, 6), _f32),
        jax.ShapeDtypeStruct((1, 6), _f32),
    ),
)


def kernel(x, edge_index, W1, b1, W2, b2, W3, b3, fcW1, fcb1, fcW2, fcb2,
           isW, isb, mcW, mcb):
    pad = jnp.full((EPAD - E,), N, _i32)
    srcr = jnp.concatenate([edge_index[0], pad]).reshape(NS, NCH, CH)
    dstr = jnp.concatenate([edge_index[1], pad]).reshape(NS, NCH, CH)
    srcc = jnp.stack([srcr, srcr + NP])

    ones_y = jnp.ones((NC * NP, HH), _f32)
    deg_col = _deg_call(ones_y, dstr)[:N, 0:1]

    y, dinv = _first_call(x, W1, deg_col)
    agg = _agg_call(y, srcc, dstr)
    y = _mid_call(agg, dinv, b1, W2)
    agg = _agg_call(y, srcc, dstr)
    y = _mid_call(agg, dinv, b2, W3)
    agg = _agg_call(y, srcc, dstr)
    return _head_call(agg, dinv, b3, x, fcW1, fcb1, fcW2, fcb2,
                      isW, isb, mcW, mcb)


# speedup vs baseline: 1.6772x; 1.0286x over previous
import functools

import jax
import jax.numpy as jnp
from jax import lax
from jax.experimental import pallas as pl
from jax.experimental.pallas import tpu as pltpu
from jax.experimental.pallas import tpu_sc as plsc

N = 10000
D = 128
H = 256
HH = H // 2
E = 320000

NC = 2
NS = 16
CH = 120
NCH = 168
BLK = 8
NBLK = NCH // BLK
NBUF = 3
EPT = NCH * CH
EPAD = EPT * NS

NP = N + 8
INITR = 632
OUTR = 632

_f32 = jnp.float32
_i32 = jnp.int32


def _sc_mesh():
    return plsc.VectorSubcoreMesh(core_axis_name="c", subcore_axis_name="s")


def _agg_body(y_hbm, src_hbm, dst_hbm, out_hbm, sidx, didx, rows0, rows1,
              rows2, acc, sem0, sem1, sem2):
    c = lax.axis_index("c")
    s = lax.axis_index("s")
    ibase = c * NP + jnp.minimum(s * INITR, NP - INITR)
    iacc = jnp.minimum(s * INITR, NP - INITR)
    pltpu.sync_copy(y_hbm.at[pl.ds(ibase, INITR)], acc.at[pl.ds(iacc, INITR)])
    plsc.subcore_barrier()

    rows = (rows0, rows1, rows2)
    sems = (sem0, sem1, sem2)

    def _block(bi, carry):
        pltpu.sync_copy(src_hbm.at[c, s, pl.ds(bi * BLK, BLK)], sidx)
        pltpu.sync_copy(dst_hbm.at[s, pl.ds(bi * BLK, BLK)], didx)
        descs = [None] * BLK
        for j in range(NBUF - 1):
            descs[j] = pltpu.async_copy(y_hbm.at[sidx.at[j]], rows[j % NBUF],
                                        sems[j % NBUF])
        for j in range(BLK):
            descs[j].wait()
            if j + NBUF - 1 < BLK:
                k = j + NBUF - 1
                descs[k] = pltpu.async_copy(y_hbm.at[sidx.at[k]],
                                            rows[k % NBUF], sems[k % NBUF])
            pltpu.sync_copy(rows[j % NBUF], acc.at[didx.at[j]], add=True)
        return carry

    lax.fori_loop(0, NBLK, _block, 0)
    plsc.subcore_barrier()
    obase = jnp.minimum(s * OUTR, N - OUTR)
    pltpu.sync_copy(acc.at[pl.ds(obase, OUTR)],
                    out_hbm.at[pl.ds(c * N + obase, OUTR)])


_agg_call = functools.partial(
    pl.kernel,
    out_type=jax.ShapeDtypeStruct((NC * N, HH), _f32),
    mesh=_sc_mesh(),
    scratch_types=[
        pltpu.VMEM((BLK, CH), _i32),
        pltpu.VMEM((BLK, CH), _i32),
        pltpu.VMEM((CH, HH), _f32),
        pltpu.VMEM((CH, HH), _f32),
        pltpu.VMEM((CH, HH), _f32),
        pltpu.VMEM_SHARED((NP, HH), _f32),
        pltpu.SemaphoreType.DMA,
        pltpu.SemaphoreType.DMA,
        pltpu.SemaphoreType.DMA,
    ],
)(_agg_body)


def _deg_body(y_hbm, dst_hbm, out_hbm, didx, rows, acc):
    c = lax.axis_index("c")
    s = lax.axis_index("s")
    ibase = c * NP + jnp.minimum(s * INITR, NP - INITR)
    iacc = jnp.minimum(s * INITR, NP - INITR)
    pltpu.sync_copy(y_hbm.at[pl.ds(ibase, INITR)], acc.at[pl.ds(iacc, INITR)])
    pltpu.sync_copy(y_hbm.at[pl.ds(0, CH)], rows)
    plsc.subcore_barrier()

    def _block(bi, carry):
        pltpu.sync_copy(dst_hbm.at[s, pl.ds(bi * BLK, BLK)], didx)

        def _chunk(j, carry2):
            pltpu.sync_copy(rows, acc.at[didx.at[j]], add=True)
            return carry2

        lax.fori_loop(0, BLK, _chunk, 0)
        return carry

    lax.fori_loop(0, NBLK, _block, 0)
    plsc.subcore_barrier()
    obase = jnp.minimum(s * OUTR, N - OUTR)
    pltpu.sync_copy(acc.at[pl.ds(obase, OUTR)],
                    out_hbm.at[pl.ds(c * N + obase, OUTR)])


_deg_call = functools.partial(
    pl.kernel,
    out_type=jax.ShapeDtypeStruct((NC * N, HH), _f32),
    mesh=_sc_mesh(),
    scratch_types=[
        pltpu.VMEM((BLK, CH), _i32),
        pltpu.VMEM((CH, HH), _f32),
        pltpu.VMEM_SHARED((NP, HH), _f32),
    ],
)(_deg_body)


def _first_body(x_ref, w_ref, deg_ref, y_ref, dinv_ref):
    dinv = lax.rsqrt(deg_ref[...])
    xw = jnp.dot(x_ref[...], w_ref[...], preferred_element_type=_f32)
    y = xw * dinv
    y_ref[:N, :] = y[:, :HH]
    y_ref[NP:NP + N, :] = y[:, HH:]
    dinv_ref[...] = dinv


_first_call = pl.pallas_call(
    _first_body,
    out_shape=(
        jax.ShapeDtypeStruct((NC * NP, HH), _f32),
        jax.ShapeDtypeStruct((N, 1), _f32),
    ),
)


def _mid_body(a_ref, dinv_ref, b_ref, w_ref, y_ref):
    dinv = dinv_ref[...]
    b = b_ref[...]
    h0 = jnp.maximum(a_ref[:N] * dinv + b[:HH], 0.0)
    h1 = jnp.maximum(a_ref[N:] * dinv + b[HH:], 0.0)
    y = (jnp.dot(h0, w_ref[:HH, :], preferred_element_type=_f32)
         + jnp.dot(h1, w_ref[HH:, :], preferred_element_type=_f32)) * dinv
    y_ref[:N, :] = y[:, :HH]
    y_ref[NP:NP + N, :] = y[:, HH:]


_mid_call = pl.pallas_call(
    _mid_body,
    out_shape=jax.ShapeDtypeStruct((NC * NP, HH), _f32),
)


def _head_body(a_ref, dinv_ref, b_ref, x_ref, fw1_ref, fb1_ref, fw2_ref,
               fb2_ref, iw_ref, ib_ref, mw_ref, mb_ref, is_ref, mc_ref):
    dinv = dinv_ref[...]
    b = b_ref[...]
    h0 = jnp.maximum(a_ref[:N] * dinv + b[:HH], 0.0)
    h1 = jnp.maximum(a_ref[N:] * dinv + b[HH:], 0.0)
    m0 = jnp.mean(h0, axis=0, keepdims=True)
    m1 = jnp.mean(h1, axis=0, keepdims=True)

    xv = x_ref[...]
    col = lambda i: xv[:, i:i + 1]
    ssum = lambda v: jnp.sum(v, axis=0, keepdims=True)
    n_comp = ssum(col(2))
    n_and = ssum(col(3))
    n_or = ssum(col(4))
    comp = col(2) == 1.0
    cnt = ssum(comp.astype(_f32))
    sum_l = ssum(jnp.where(comp, col(0), 0.0))
    sum_m = ssum(jnp.where(comp, col(1), 0.0))
    avg_l = jnp.where(cnt > 0, sum_l / jnp.maximum(cnt, 1.0), 0.0)
    avg_m = jnp.where(cnt > 0, sum_m / jnp.maximum(cnt, 1.0), 0.0)
    t_norm = jnp.full((1, 1), 100.0 / 500.0, _f32)
    t_fac = jnp.full((1, 1), 1.0 + 6.0 * (1.0 - 100.0 / 500.0) ** 1.5, _f32)
    gf = jnp.concatenate(
        [n_comp, n_and, n_or, n_and + n_or, avg_l, avg_m, t_norm, t_fac],
        axis=1)

    emb = jnp.concatenate([m0, m1, gf], axis=1)
    f = jnp.maximum(
        jnp.dot(emb, fw1_ref[...], preferred_element_type=_f32) + fb1_ref[...],
        0.0)
    f = jnp.maximum(
        jnp.dot(f, fw2_ref[...], preferred_element_type=_f32) + fb2_ref[...],
        0.0)
    is_ref[...] = jnp.dot(f, iw_ref[...], preferred_element_type=_f32) + ib_ref[...]
    mc_ref[...] = jnp.dot(f, mw_ref[...], preferred_element_type=_f32) + mb_ref[...]


_head_call = pl.pallas_call(
    _head_body,
    out_shape=(
        jax.ShapeDtypeStruct((1, 6), _f32),
        jax.ShapeDtypeStruct((1, 6), _f32),
    ),
)


def kernel(x, edge_index, W1, b1, W2, b2, W3, b3, fcW1, fcb1, fcW2, fcb2,
           isW, isb, mcW, mcb):
    pad = jnp.full((EPAD - E,), N, _i32)
    srcr = jnp.concatenate([edge_index[0], pad]).reshape(NS, NCH, CH)
    dstr = jnp.concatenate([edge_index[1], pad]).reshape(NS, NCH, CH)
    srcc = jnp.stack([srcr, srcr + NP])

    ones_y = jnp.ones((NC * NP, HH), _f32)
    deg_col = _deg_call(ones_y, dstr)[:N, 0:1]

    y, dinv = _first_call(x, W1, deg_col)
    agg = _agg_call(y, srcc, dstr)
    y = _mid_call(agg, dinv, b1, W2)
    agg = _agg_call(y, srcc, dstr)
    y = _mid_call(agg, dinv, b2, W3)
    agg = _agg_call(y, srcc, dstr)
    return _head_call(agg, dinv, b3, x, fcW1, fcb1, fcW2, fcb2,
                      isW, isb, mcW, mcb)
